# Initial kernel scaffold; baseline (speedup 1.0000x reference)
import functools

import jax
import jax.numpy as jnp
from jax import lax
from jax.experimental import pallas as pl
from jax.experimental.pallas import tpu as pltpu
from jax.experimental.pallas import tpu_sc as plsc

_VOCAB = 100000
_DIM = 128
_BATCH = 16384
_NEG = 20

_LANES = 16
_NW = 32
_EPW = _BATCH // _NW
_C = 32
_NCHUNK = _EPW // _C
_NROWS = _C * _NEG // 128
_HREG = _DIM // _LANES


def _sc_body(ci_hbm, co_hbm, ns_hbm, cw_hbm, xw_hbm,
             score_hbm, negdot_hbm,
             nsidx, ci_idx, co_idx, negbuf, v_buf, u_buf, sbuf, nbuf, sem):
    nc = 2
    wid = lax.axis_index("s") * nc + lax.axis_index("c")

    def chunk_body(c, carry):
        widc = wid * _NCHUNK + c
        base = widc * _C
        pltpu.sync_copy(ci_hbm.at[widc], ci_idx)
        pltpu.sync_copy(co_hbm.at[widc], co_idx)
        pltpu.sync_copy(ns_hbm.at[widc], nsidx)
        cps = []
        for j in range(_NROWS):
            cps.append(pltpu.async_copy(
                xw_hbm.at[nsidx.at[j]], negbuf.at[pl.ds(j * 128, 128)], sem))
        cps.append(pltpu.async_copy(cw_hbm.at[ci_idx], v_buf, sem))
        cps.append(pltpu.async_copy(xw_hbm.at[co_idx], u_buf, sem))
        for cp in cps:
            cp.wait()

        def elem_body(b, inner):
            r0 = b * _NEG
            accs = [negbuf[r0, pl.ds(h * _LANES, _LANES)] for h in range(_HREG)]
            for k in range(1, _NEG):
                for h in range(_HREG):
                    accs[h] = accs[h] + negbuf[r0 + k, pl.ds(h * _LANES, _LANES)]
            sd = None
            nd = None
            for h in range(_HREG):
                vv = v_buf[b, pl.ds(h * _LANES, _LANES)]
                uu = u_buf[b, pl.ds(h * _LANES, _LANES)]
                sd = vv * uu if sd is None else sd + vv * uu
                nd = vv * accs[h] if nd is None else nd + vv * accs[h]
            sbuf[b] = sd
            nbuf[b] = nd
            return inner

        lax.fori_loop(0, _C, elem_body, 0)
        pltpu.sync_copy(sbuf, score_hbm.at[pl.ds(base, _C)])
        pltpu.sync_copy(nbuf, negdot_hbm.at[pl.ds(base, _C)])
        return carry

    lax.fori_loop(0, _NCHUNK, chunk_body, 0)


_sc_call = pl.kernel(
    _sc_body,
    out_type=[
        jax.ShapeDtypeStruct((_BATCH, _LANES), jnp.float32),
        jax.ShapeDtypeStruct((_BATCH, _LANES), jnp.float32),
    ],
    mesh=plsc.VectorSubcoreMesh(core_axis_name="c", subcore_axis_name="s"),
    scratch_types=[
        pltpu.VMEM((_NROWS, 128), jnp.int32),
        pltpu.VMEM((_C,), jnp.int32),
        pltpu.VMEM((_C,), jnp.int32),
        pltpu.VMEM((_C * _NEG, _DIM), jnp.float32),
        pltpu.VMEM((_C, _DIM), jnp.float32),
        pltpu.VMEM((_C, _DIM), jnp.float32),
        pltpu.VMEM((_C, _LANES), jnp.float32),
        pltpu.VMEM((_C, _LANES), jnp.float32),
        pltpu.SemaphoreType.DMA,
    ],
)


def _log_sigmoid(x):
    return jnp.minimum(x, 0.0) - jnp.log1p(jnp.exp(-jnp.abs(x)))


def _tc_body(sp_ref, np_ref, out_ref):
    s = jnp.sum(sp_ref[...], axis=1)
    n = jnp.sum(np_ref[...], axis=1)
    loss = _log_sigmoid(s) + _log_sigmoid(-n)
    out_ref[0, 0] = -jnp.mean(loss)


_tc_reduce = pl.pallas_call(
    _tc_body,
    out_shape=jax.ShapeDtypeStruct((1, 1), jnp.float32),
)


def kernel(center_input, context_output, negative_samples,
           center_weight, context_weight):
    ci_r = center_input.reshape(_NW * _NCHUNK, _C)
    co_r = context_output.reshape(_NW * _NCHUNK, _C)
    ns_r = negative_samples.reshape(_NW * _NCHUNK, _NROWS, 128)
    score_p, negdot_p = _sc_call(ci_r, co_r, ns_r, center_weight, context_weight)
    res = _tc_reduce(score_p, negdot_p)
    return res[0, 0]


# speedup vs baseline: 6.5464x; 6.5464x over previous
import functools

import jax
import jax.numpy as jnp
from jax import lax
from jax.experimental import pallas as pl
from jax.experimental.pallas import tpu as pltpu
from jax.experimental.pallas import tpu_sc as plsc

_VOCAB = 100000
_DIM = 128
_BATCH = 16384
_NEG = 20

_LANES = 16
_NW = 32
_EPW = _BATCH // _NW
_C = 32
_NCHUNK = _EPW // _C
_NROWS = _C * _NEG // 128
_HREG = _DIM // _LANES


def _sc_body(ci_hbm, co_hbm, ns_hbm, cw_hbm, xw_hbm,
             score_hbm, negdot_hbm,
             nsidx, ci_idx, co_idx, negbuf, v_buf, u_buf, sbuf, nbuf, sem):
    nc = 2
    wid = lax.axis_index("s") * nc + lax.axis_index("c")

    def chunk_body(c, carry):
        widc = wid * _NCHUNK + c
        base = widc * _C
        pltpu.sync_copy(ci_hbm.at[widc], ci_idx)
        pltpu.sync_copy(co_hbm.at[widc], co_idx)
        pltpu.sync_copy(ns_hbm.at[widc], nsidx)
        cps = []
        for j in range(_NROWS):
            cps.append(pltpu.async_copy(
                xw_hbm.at[nsidx.at[j]], negbuf.at[pl.ds(j * 128, 128)], sem))
        cps.append(pltpu.async_copy(cw_hbm.at[ci_idx], v_buf, sem))
        cps.append(pltpu.async_copy(xw_hbm.at[co_idx], u_buf, sem))
        for cp in cps:
            cp.wait()

        def elem_body(b, inner):
            r0 = b * _NEG
            accs = [negbuf[r0, pl.ds(h * _LANES, _LANES)] for h in range(_HREG)]
            for k in range(1, _NEG):
                for h in range(_HREG):
                    accs[h] = accs[h] + negbuf[r0 + k, pl.ds(h * _LANES, _LANES)]
            sd = None
            nd = None
            for h in range(_HREG):
                vv = v_buf[b, pl.ds(h * _LANES, _LANES)]
                uu = u_buf[b, pl.ds(h * _LANES, _LANES)]
                sd = vv * uu if sd is None else sd + vv * uu
                nd = vv * accs[h] if nd is None else nd + vv * accs[h]
            sbuf[b] = sd
            nbuf[b] = nd
            return inner

        lax.fori_loop(0, _C, elem_body, 0)
        pltpu.sync_copy(sbuf, score_hbm.at[pl.ds(base, _C)])
        pltpu.sync_copy(nbuf, negdot_hbm.at[pl.ds(base, _C)])
        return carry

    lax.fori_loop(0, _NCHUNK, chunk_body, 0)


_sc_call = pl.kernel(
    _sc_body,
    out_type=[
        jax.ShapeDtypeStruct((_BATCH, _LANES), jnp.float32),
        jax.ShapeDtypeStruct((_BATCH, _LANES), jnp.float32),
    ],
    mesh=plsc.VectorSubcoreMesh(core_axis_name="c", subcore_axis_name="s"),
    scratch_types=[
        pltpu.VMEM((_NROWS, 128), jnp.int32),
        pltpu.VMEM((_C,), jnp.int32),
        pltpu.VMEM((_C,), jnp.int32),
        pltpu.VMEM((_C * _NEG, _DIM), jnp.float32),
        pltpu.VMEM((_C, _DIM), jnp.float32),
        pltpu.VMEM((_C, _DIM), jnp.float32),
        pltpu.VMEM((_C, _LANES), jnp.float32),
        pltpu.VMEM((_C, _LANES), jnp.float32),
        pltpu.SemaphoreType.DMA,
    ],
)


def _log_sigmoid(x):
    return jnp.minimum(x, 0.0) - jnp.log1p(jnp.exp(-jnp.abs(x)))


def _tc_body(sp_ref, np_ref, out_ref):
    s = jnp.sum(sp_ref[...], axis=1)
    n = jnp.sum(np_ref[...], axis=1)
    loss = _log_sigmoid(s) + _log_sigmoid(-n)
    out_ref[...] = jnp.reshape(-jnp.mean(loss), (1, 1))


_tc_reduce = pl.pallas_call(
    _tc_body,
    out_shape=jax.ShapeDtypeStruct((1, 1), jnp.float32),
)


def kernel(center_input, context_output, negative_samples,
           center_weight, context_weight):
    ci_r = center_input.reshape(_NW * _NCHUNK, _C)
    co_r = context_output.reshape(_NW * _NCHUNK, _C)
    ns_r = negative_samples.reshape(_NW * _NCHUNK, _NROWS, 128)
    score_p, negdot_p = _sc_call(ci_r, co_r, ns_r, center_weight, context_weight)
    res = _tc_reduce(score_p, negdot_p)
    return res[0, 0]


# speedup vs baseline: 10.6791x; 1.6313x over previous
import jax
import jax.numpy as jnp
from jax import lax
from jax.experimental import pallas as pl
from jax.experimental.pallas import tpu as pltpu
from jax.experimental.pallas import tpu_sc as plsc

_VOCAB = 100000
_DIM = 128
_BATCH = 16384
_NEG = 20

_LANES = 16
_NW = 32
_EPW = _BATCH // _NW
_C = 16
_NCHUNK = _EPW // _C
_G = 5
_IW = _C * _NEG // _G
_HREG = _DIM // _LANES
_NPAIR = _NCHUNK // 2


def _sc_body(ci_hbm, co_hbm, ns_hbm, cw_hbm, xw_hbm,
             score_hbm, negdot_hbm,
             ci_all, co_all, ns_all,
             negbuf0, negbuf1, v0, v1, u0, u1,
             sb0, sb1, nb0, nb1,
             sg0, sg1, so0, so1):
    nc = 2
    wid = lax.axis_index("s") * nc + lax.axis_index("c")

    def mkcopies(c, negbuf, vbuf, ubuf, sem):
        cps = []
        for j in range(_G):
            cps.append(pltpu.make_async_copy(
                xw_hbm.at[ns_all.at[c * _G + j]],
                negbuf.at[pl.ds(j * _IW, _IW)], sem))
        cps.append(pltpu.make_async_copy(
            cw_hbm.at[ci_all.at[pl.ds(c * _C, _C)]], vbuf, sem))
        cps.append(pltpu.make_async_copy(
            xw_hbm.at[co_all.at[pl.ds(c * _C, _C)]], ubuf, sem))
        return cps

    def fire(c, negbuf, vbuf, ubuf, sem):
        for cp in mkcopies(c, negbuf, vbuf, ubuf, sem):
            cp.start()

    def compute(negbuf, vbuf, ubuf, sbuf, nbuf):
        def elem_body(b, inner):
            r0 = b * _NEG
            accs = [negbuf[r0, pl.ds(h * _LANES, _LANES)] for h in range(_HREG)]
            for k in range(1, _NEG):
                for h in range(_HREG):
                    accs[h] = accs[h] + negbuf[r0 + k, pl.ds(h * _LANES, _LANES)]
            sd = None
            nd = None
            for h in range(_HREG):
                vv = vbuf[b, pl.ds(h * _LANES, _LANES)]
                uu = ubuf[b, pl.ds(h * _LANES, _LANES)]
                sd = vv * uu if sd is None else sd + vv * uu
                nd = vv * accs[h] if nd is None else nd + vv * accs[h]
            sbuf[b] = sd
            nbuf[b] = nd
            return inner

        lax.fori_loop(0, _C, elem_body, 0)

    pltpu.sync_copy(ci_hbm.at[wid], ci_all)
    pltpu.sync_copy(co_hbm.at[wid], co_all)
    pltpu.sync_copy(ns_hbm.at[wid], ns_all)
    fire(0, negbuf0, v0, u0, sg0)
    fire(1, negbuf1, v1, u1, sg1)

    bufs = (
        (negbuf0, v0, u0, sg0, so0, sb0, nb0),
        (negbuf1, v1, u1, sg1, so1, sb1, nb1),
    )

    def pair_body(g, carry):
        for p, (negbuf, vbuf, ubuf, sgsem, sosem, sbuf, nbuf) in enumerate(bufs):
            c = g * 2 + p
            base = wid * _EPW + c * _C
            for cp in mkcopies(c, negbuf, vbuf, ubuf, sgsem):
                cp.wait()

            @pl.when(g >= 1)
            def _():
                bm2 = base - 2 * _C
                pltpu.make_async_copy(
                    sbuf, score_hbm.at[pl.ds(bm2, _C)], sosem).wait()
                pltpu.make_async_copy(
                    nbuf, negdot_hbm.at[pl.ds(bm2, _C)], sosem).wait()

            compute(negbuf, vbuf, ubuf, sbuf, nbuf)
            pltpu.async_copy(sbuf, score_hbm.at[pl.ds(base, _C)], sosem)
            pltpu.async_copy(nbuf, negdot_hbm.at[pl.ds(base, _C)], sosem)

            @pl.when(g < _NPAIR - 1)
            def _():
                fire(c + 2, negbuf, vbuf, ubuf, sgsem)
        return carry

    lax.fori_loop(0, _NPAIR, pair_body, 0)

    for p, (_, _, _, _, sosem, sbuf, nbuf) in enumerate(bufs):
        c = _NCHUNK - 2 + p
        base = wid * _EPW + c * _C
        pltpu.make_async_copy(sbuf, score_hbm.at[pl.ds(base, _C)], sosem).wait()
        pltpu.make_async_copy(nbuf, negdot_hbm.at[pl.ds(base, _C)], sosem).wait()


_sc_call = pl.kernel(
    _sc_body,
    out_type=[
        jax.ShapeDtypeStruct((_BATCH, _LANES), jnp.float32),
        jax.ShapeDtypeStruct((_BATCH, _LANES), jnp.float32),
    ],
    mesh=plsc.VectorSubcoreMesh(core_axis_name="c", subcore_axis_name="s"),
    scratch_types=[
        pltpu.VMEM((_EPW,), jnp.int32),
        pltpu.VMEM((_EPW,), jnp.int32),
        pltpu.VMEM((_NCHUNK * _G, _IW), jnp.int32),
        pltpu.VMEM((_C * _NEG, _DIM), jnp.float32),
        pltpu.VMEM((_C * _NEG, _DIM), jnp.float32),
        pltpu.VMEM((_C, _DIM), jnp.float32),
        pltpu.VMEM((_C, _DIM), jnp.float32),
        pltpu.VMEM((_C, _DIM), jnp.float32),
        pltpu.VMEM((_C, _DIM), jnp.float32),
        pltpu.VMEM((_C, _LANES), jnp.float32),
        pltpu.VMEM((_C, _LANES), jnp.float32),
        pltpu.VMEM((_C, _LANES), jnp.float32),
        pltpu.VMEM((_C, _LANES), jnp.float32),
        pltpu.SemaphoreType.DMA,
        pltpu.SemaphoreType.DMA,
        pltpu.SemaphoreType.DMA,
        pltpu.SemaphoreType.DMA,
    ],
)


def _log_sigmoid(x):
    return jnp.minimum(x, 0.0) - jnp.log1p(jnp.exp(-jnp.abs(x)))


def _tc_body(sp_ref, np_ref, out_ref):
    s = jnp.sum(sp_ref[...], axis=1)
    n = jnp.sum(np_ref[...], axis=1)
    loss = _log_sigmoid(s) + _log_sigmoid(-n)
    out_ref[...] = jnp.reshape(-jnp.mean(loss), (1, 1))


_tc_reduce = pl.pallas_call(
    _tc_body,
    out_shape=jax.ShapeDtypeStruct((1, 1), jnp.float32),
)


def kernel(center_input, context_output, negative_samples,
           center_weight, context_weight):
    ci_r = center_input.reshape(_NW, _EPW)
    co_r = context_output.reshape(_NW, _EPW)
    ns_r = negative_samples.reshape(_NW, _NCHUNK * _G, _IW)
    score_p, negdot_p = _sc_call(ci_r, co_r, ns_r, center_weight, context_weight)
    res = _tc_reduce(score_p, negdot_p)
    return res[0, 0]


# speedup vs baseline: 12.3538x; 1.1568x over previous
import jax
import jax.numpy as jnp
from jax import lax
from jax.experimental import pallas as pl
from jax.experimental.pallas import tpu as pltpu
from jax.experimental.pallas import tpu_sc as plsc

_VOCAB = 100000
_DIM = 128
_BATCH = 16384
_NEG = 20

_LANES = 16
_NW = 32
_EPW = _BATCH // _NW
_C = 64
_NCHUNK = _EPW // _C
_HREG = _DIM // _LANES
_NPAIR = _NCHUNK // 2


def _sc_body(ci_hbm, co_hbm, ns_hbm, cw_hbm, xw_hbm,
             score_hbm, negdot_hbm,
             ci_all, co_all, ns_all,
             nsum0, nsum1, v0, v1, u0, u1,
             sb0, sb1, nb0, nb1,
             sg0, sg1, so0, so1):
    nc = 2
    wid = lax.axis_index("s") * nc + lax.axis_index("c")

    def mkcopies(c, nsum, vbuf, ubuf, sem):
        cps = []
        for k in range(_NEG):
            cps.append((pltpu.make_async_copy(
                xw_hbm.at[ns_all.at[c * _NEG + k]], nsum, sem), True))
        cps.append((pltpu.make_async_copy(
            cw_hbm.at[ci_all.at[pl.ds(c * _C, _C)]], vbuf, sem), False))
        cps.append((pltpu.make_async_copy(
            xw_hbm.at[co_all.at[pl.ds(c * _C, _C)]], ubuf, sem), False))
        return cps

    def zero_and_fire(c, nsum, vbuf, ubuf, sem):
        zeros = jnp.zeros((_LANES,), jnp.float32)

        def zrow(b, inner):
            for h in range(_HREG):
                nsum[b, pl.ds(h * _LANES, _LANES)] = zeros
            return inner

        lax.fori_loop(0, _C, zrow, 0)
        for cp, add in mkcopies(c, nsum, vbuf, ubuf, sem):
            cp.start(add=add)

    def compute(nsum, vbuf, ubuf, sbuf, nbuf):
        def elem_body(b, inner):
            sd = None
            nd = None
            for h in range(_HREG):
                vv = vbuf[b, pl.ds(h * _LANES, _LANES)]
                uu = ubuf[b, pl.ds(h * _LANES, _LANES)]
                nn = nsum[b, pl.ds(h * _LANES, _LANES)]
                sd = vv * uu if sd is None else sd + vv * uu
                nd = vv * nn if nd is None else nd + vv * nn
            sbuf[b] = sd
            nbuf[b] = nd
            return inner

        lax.fori_loop(0, _C, elem_body, 0)

    pltpu.sync_copy(ci_hbm.at[wid], ci_all)
    pltpu.sync_copy(co_hbm.at[wid], co_all)
    pltpu.sync_copy(ns_hbm.at[wid], ns_all)
    zero_and_fire(0, nsum0, v0, u0, sg0)
    zero_and_fire(1, nsum1, v1, u1, sg1)

    bufs = (
        (nsum0, v0, u0, sg0, so0, sb0, nb0),
        (nsum1, v1, u1, sg1, so1, sb1, nb1),
    )

    def pair_body(g, carry):
        for p, (nsum, vbuf, ubuf, sgsem, sosem, sbuf, nbuf) in enumerate(bufs):
            c = g * 2 + p
            base = wid * _EPW + c * _C
            for cp, _unused in mkcopies(c, nsum, vbuf, ubuf, sgsem):
                cp.wait()

            @pl.when(g >= 1)
            def _():
                bm2 = base - 2 * _C
                pltpu.make_async_copy(
                    sbuf, score_hbm.at[pl.ds(bm2, _C)], sosem).wait()
                pltpu.make_async_copy(
                    nbuf, negdot_hbm.at[pl.ds(bm2, _C)], sosem).wait()

            compute(nsum, vbuf, ubuf, sbuf, nbuf)
            pltpu.async_copy(sbuf, score_hbm.at[pl.ds(base, _C)], sosem)
            pltpu.async_copy(nbuf, negdot_hbm.at[pl.ds(base, _C)], sosem)

            @pl.when(g < _NPAIR - 1)
            def _():
                zero_and_fire(c + 2, nsum, vbuf, ubuf, sgsem)
        return carry

    lax.fori_loop(0, _NPAIR, pair_body, 0)

    for p, (_, _, _, _, sosem, sbuf, nbuf) in enumerate(bufs):
        c = _NCHUNK - 2 + p
        base = wid * _EPW + c * _C
        pltpu.make_async_copy(sbuf, score_hbm.at[pl.ds(base, _C)], sosem).wait()
        pltpu.make_async_copy(nbuf, negdot_hbm.at[pl.ds(base, _C)], sosem).wait()


_sc_call = pl.kernel(
    _sc_body,
    out_type=[
        jax.ShapeDtypeStruct((_BATCH, _LANES), jnp.float32),
        jax.ShapeDtypeStruct((_BATCH, _LANES), jnp.float32),
    ],
    mesh=plsc.VectorSubcoreMesh(core_axis_name="c", subcore_axis_name="s"),
    scratch_types=[
        pltpu.VMEM((_EPW,), jnp.int32),
        pltpu.VMEM((_EPW,), jnp.int32),
        pltpu.VMEM((_NCHUNK * _NEG, _C), jnp.int32),
        pltpu.VMEM((_C, _DIM), jnp.float32),
        pltpu.VMEM((_C, _DIM), jnp.float32),
        pltpu.VMEM((_C, _DIM), jnp.float32),
        pltpu.VMEM((_C, _DIM), jnp.float32),
        pltpu.VMEM((_C, _DIM), jnp.float32),
        pltpu.VMEM((_C, _DIM), jnp.float32),
        pltpu.VMEM((_C, _LANES), jnp.float32),
        pltpu.VMEM((_C, _LANES), jnp.float32),
        pltpu.VMEM((_C, _LANES), jnp.float32),
        pltpu.VMEM((_C, _LANES), jnp.float32),
        pltpu.SemaphoreType.DMA,
        pltpu.SemaphoreType.DMA,
        pltpu.SemaphoreType.DMA,
        pltpu.SemaphoreType.DMA,
    ],
)


def _log_sigmoid(x):
    return jnp.minimum(x, 0.0) - jnp.log1p(jnp.exp(-jnp.abs(x)))


def _tc_body(sp_ref, np_ref, out_ref):
    s = jnp.sum(sp_ref[...], axis=1)
    n = jnp.sum(np_ref[...], axis=1)
    loss = _log_sigmoid(s) + _log_sigmoid(-n)
    out_ref[...] = jnp.reshape(-jnp.mean(loss), (1, 1))


_tc_reduce = pl.pallas_call(
    _tc_body,
    out_shape=jax.ShapeDtypeStruct((1, 1), jnp.float32),
)


def kernel(center_input, context_output, negative_samples,
           center_weight, context_weight):
    ci_r = center_input.reshape(_NW, _EPW)
    co_r = context_output.reshape(_NW, _EPW)
    ns_r = (negative_samples
            .reshape(_NW, _NCHUNK, _C, _NEG)
            .transpose(0, 1, 3, 2)
            .reshape(_NW, _NCHUNK * _NEG, _C))
    score_p, negdot_p = _sc_call(ci_r, co_r, ns_r, center_weight, context_weight)
    res = _tc_reduce(score_p, negdot_p)
    return res[0, 0]
